# baseline (speedup 1.0000x reference)
import functools

import jax
import jax.numpy as jnp
import numpy as np
from jax import lax
from jax.experimental import pallas as pl
from jax.experimental.pallas import tpu as pltpu
from jax.experimental.pallas import tpu_sc as plsc

_DIM_USED = np.array(
    [6, 7, 8, 9, 10, 11, 12, 13, 14, 15, 16, 17, 21, 22, 23, 24, 25, 26,
     27, 28, 29, 30, 31, 32, 36, 37, 38, 39, 40, 41, 42, 43, 44, 45, 46,
     47, 51, 52, 53, 54, 55, 56, 57, 58, 59, 63, 64, 65, 66, 67, 68, 75,
     76, 77, 78, 79, 80, 81, 82, 83, 87, 88, 89, 90, 91, 92],
    dtype=np.int32,
)

D_IN = 96
D_OUT = 66
N_ROWS = 4096 * 50
NC, NS = 2, 16
NW = NC * NS
ROWS_PER_TILE = N_ROWS // NW
CHUNK_ROWS = 640
NCHUNK = ROWS_PER_TILE // CHUNK_ROWS
BLOCKS = CHUNK_ROWS // 8
NVEC = 8 * D_OUT // 16

_p = np.arange(8 * D_OUT)
_TAB = ((_p // D_OUT) * D_IN + _DIM_USED[_p % D_OUT]).astype(np.int32)
_TAB = _TAB.reshape(NVEC, 16)


@functools.partial(
    pl.kernel,
    out_type=jax.ShapeDtypeStruct((N_ROWS * D_OUT,), jnp.float32),
    mesh=plsc.VectorSubcoreMesh(core_axis_name="c", subcore_axis_name="s"),
    scratch_types=[
        pltpu.VMEM((NVEC, 16), jnp.int32),
        pltpu.VMEM((CHUNK_ROWS * D_IN,), jnp.float32),
        pltpu.VMEM((CHUNK_ROWS * D_OUT,), jnp.float32),
    ],
    compiler_params=pltpu.CompilerParams(needs_layout_passes=False),
)
def _sc_select(x_hbm, tab_hbm, out_hbm, tab_v, in_v, out_v):
    wid = lax.axis_index("s") * NC + lax.axis_index("c")
    pltpu.sync_copy(tab_hbm, tab_v)
    base = wid * ROWS_PER_TILE

    def chunk_body(c, carry):
        r0 = base + c * CHUNK_ROWS
        pltpu.sync_copy(x_hbm.at[pl.ds(r0 * D_IN, CHUNK_ROWS * D_IN)], in_v)
        idx0 = tuple(tab_v[j, :] for j in range(NVEC))

        def blk(b, idxs):
            ob = b * (8 * D_OUT)
            for j in range(NVEC):
                v = plsc.load_gather(in_v, [idxs[j]])
                out_v[pl.ds(ob + j * 16, 16)] = v
            return tuple(i + 8 * D_IN for i in idxs)

        lax.fori_loop(0, BLOCKS, blk, idx0)
        pltpu.sync_copy(out_v, out_hbm.at[pl.ds(r0 * D_OUT, CHUNK_ROWS * D_OUT)])
        return carry

    lax.fori_loop(0, NCHUNK, chunk_body, 0)


def kernel(observed_pose):
    x = observed_pose.reshape(N_ROWS * D_IN)
    out = _sc_select(x, jnp.asarray(_TAB))
    return out.reshape(4096, 50, D_OUT)


# speedup vs baseline: 1.0726x; 1.0726x over previous
import functools

import jax
import jax.numpy as jnp
import numpy as np
from jax import lax
from jax.experimental import pallas as pl
from jax.experimental.pallas import tpu as pltpu
from jax.experimental.pallas import tpu_sc as plsc

_DIM_USED = np.array(
    [6, 7, 8, 9, 10, 11, 12, 13, 14, 15, 16, 17, 21, 22, 23, 24, 25, 26,
     27, 28, 29, 30, 31, 32, 36, 37, 38, 39, 40, 41, 42, 43, 44, 45, 46,
     47, 51, 52, 53, 54, 55, 56, 57, 58, 59, 63, 64, 65, 66, 67, 68, 75,
     76, 77, 78, 79, 80, 81, 82, 83, 87, 88, 89, 90, 91, 92],
    dtype=np.int32,
)

D_IN = 96
D_OUT = 66
N_ROWS = 4096 * 50
NC, NS = 2, 16
NW = NC * NS
ROWS_PER_TILE = N_ROWS // NW
CHUNK_ROWS = 320
NCHUNK = ROWS_PER_TILE // CHUNK_ROWS
BLOCKS = CHUNK_ROWS // 8
NVEC = 8 * D_OUT // 16

_p = np.arange(8 * D_OUT)
_TAB = ((_p // D_OUT) * D_IN + _DIM_USED[_p % D_OUT]).astype(np.int32)
_TAB = _TAB.reshape(NVEC, 16)


@functools.partial(
    pl.kernel,
    out_type=jax.ShapeDtypeStruct((N_ROWS * D_OUT,), jnp.float32),
    mesh=plsc.VectorSubcoreMesh(core_axis_name="c", subcore_axis_name="s"),
    scratch_types=[
        pltpu.VMEM((NVEC, 16), jnp.int32),
        pltpu.VMEM((CHUNK_ROWS * D_IN,), jnp.float32),
        pltpu.VMEM((CHUNK_ROWS * D_IN,), jnp.float32),
        pltpu.VMEM((CHUNK_ROWS * D_OUT,), jnp.float32),
        pltpu.VMEM((CHUNK_ROWS * D_OUT,), jnp.float32),
        pltpu.SemaphoreType.DMA,
        pltpu.SemaphoreType.DMA,
        pltpu.SemaphoreType.DMA,
        pltpu.SemaphoreType.DMA,
    ],
    compiler_params=pltpu.CompilerParams(needs_layout_passes=False),
)
def _sc_select(x_hbm, tab_hbm, out_hbm, tab_v, in_v0, in_v1, out_v0, out_v1,
               si0, si1, so0, so1):
    wid = lax.axis_index("s") * NC + lax.axis_index("c")
    pltpu.sync_copy(tab_hbm, tab_v)
    base = wid * ROWS_PER_TILE
    in_v = (in_v0, in_v1)
    out_v = (out_v0, out_v1)
    sin = (si0, si1)
    sout = (so0, so1)

    def start_in(c):
        r0 = base + c * CHUNK_ROWS
        return pltpu.async_copy(
            x_hbm.at[pl.ds(r0 * D_IN, CHUNK_ROWS * D_IN)],
            in_v[c % 2], sin[c % 2])

    def start_out(c):
        r0 = base + c * CHUNK_ROWS
        return pltpu.async_copy(
            out_v[c % 2],
            out_hbm.at[pl.ds(r0 * D_OUT, CHUNK_ROWS * D_OUT)], sout[c % 2])

    def repack(c):
        src = in_v[c % 2]
        dst = out_v[c % 2]
        idx0 = tuple(tab_v[j, :] for j in range(NVEC))

        def blk(b, idxs):
            ob = b * (8 * D_OUT)
            for j in range(NVEC):
                dst[pl.ds(ob + j * 16, 16)] = plsc.load_gather(src, [idxs[j]])
            return tuple(i + 8 * D_IN for i in idxs)

        lax.fori_loop(0, BLOCKS, blk, idx0)

    din = [None, None]
    dout = [None, None]
    din[0] = start_in(0)
    for c in range(NCHUNK):
        if c + 1 < NCHUNK:
            din[(c + 1) % 2] = start_in(c + 1)
        din[c % 2].wait()
        if c >= 2:
            dout[c % 2].wait()
        repack(c)
        dout[c % 2] = start_out(c)
    dout[0].wait()
    dout[1].wait()


def kernel(observed_pose):
    x = observed_pose.reshape(N_ROWS * D_IN)
    out = _sc_select(x, jnp.asarray(_TAB))
    return out.reshape(4096, 50, D_OUT)
